# baseline (speedup 1.0000x reference)
import functools

import jax
import jax.numpy as jnp
import numpy as np
from jax.experimental import pallas as pl
from jax.experimental.pallas import tpu as pltpu

N_NODES = 256
F = 128
NUM_SPECIES = 5
TDIM = 32
R_MAX = 5.0
N_LAYERS = 2
AVG_NEI = 255.0
N_RADIAL = 8

BR = 16
NB = N_NODES // BR

_C1 = float(np.sqrt(3.0))
_S15 = float(np.sqrt(15.0))
_S5 = float(np.sqrt(5.0))
_S70 = float(np.sqrt(70.0))
_S105 = float(np.sqrt(105.0))
_S42 = float(np.sqrt(42.0))
_S7 = float(np.sqrt(7.0))


def _mm(a, b):
    return jax.lax.dot_general(a, b, (((1,), (0,)), ((), ())),
                               preferred_element_type=jnp.float32,
                               precision=jax.lax.Precision.HIGHEST)


def _forward_body(pos_ref, emb_ref, wemb_ref, bemb_ref, kvec_ref, *rest):
    nparams = 12 * N_LAYERS
    layer_refs = rest[:nparams]
    out_ref = rest[nparams]
    (aggs_ref, aggvx_ref, aggvy_ref, aggvz_ref, fc0_ref,
     pcol_ref) = rest[nparams + 1:]

    f32 = jnp.float32

    px_c = pos_ref[:, 0:1]
    py_c = pos_ref[:, 1:2]
    pz_c = pos_ref[:, 2:3]
    px_r = jnp.transpose(px_c, (1, 0))
    py_r = jnp.transpose(py_c, (1, 0))
    pz_r = jnp.transpose(pz_c, (1, 0))
    px0_c, py0_c, pz0_c = px_c, py_c, pz_c

    vx0 = px_c - px_r
    vy0 = py_c - py_r
    vz0 = pz_c - pz_r
    l0 = jnp.sqrt(vx0 * vx0 + vy0 * vy0 + vz0 * vz0)
    fc = 0.5 * (jnp.cos((jnp.pi / R_MAX) * jnp.clip(l0, 0.0, R_MAX)) + 1.0)
    fc = fc * (l0 < R_MAX).astype(f32)
    ii = jax.lax.broadcasted_iota(jnp.int32, (N_NODES, N_NODES), 0)
    jj = jax.lax.broadcasted_iota(jnp.int32, (N_NODES, N_NODES), 1)
    fc0_ref[...] = jnp.where(ii == jj, 0.0, fc)

    h_s = _mm(emb_ref[...], wemb_ref[...]) + bemb_ref[...]
    h_vx = jnp.zeros((N_NODES, F), f32)
    h_vy = jnp.zeros((N_NODES, F), f32)
    h_vz = jnp.zeros((N_NODES, F), f32)

    del kvec_ref

    for l in range(N_LAYERS):
        (wr1_ref, br1_ref, wr2_ref, wsh_ref, ws_ref, wv_ref, p1_ref, p2_ref,
         p3_ref, pv_ref, wg_ref, wvec_ref) = layer_refs[12 * l:12 * (l + 1)]

        hsrc = _mm(h_s, ws_ref[...])
        hvxs = _mm(h_vx, wv_ref[...])
        hvys = _mm(h_vy, wv_ref[...])
        hvzs = _mm(h_vz, wv_ref[...])
        wr1 = wr1_ref[...]
        br1 = br1_ref[...]
        wr2 = wr2_ref[...]
        wsh = wsh_ref[...]
        pcol_ref[:, 0:1] = px_c
        pcol_ref[:, 1:2] = py_c
        pcol_ref[:, 2:3] = pz_c

        def rb_body(rb, _, px_r=px_r, py_r=py_r, pz_r=pz_r, hsrc=hsrc,
                    hvxs=hvxs, hvys=hvys, hvzs=hvzs, wr1=wr1, br1=br1,
                    wr2=wr2, wsh=wsh):
            r0 = rb * BR
            pxr = pcol_ref[pl.ds(r0, BR), 0:1]
            pyr = pcol_ref[pl.ds(r0, BR), 1:2]
            pzr = pcol_ref[pl.ds(r0, BR), 2:3]
            vx = pxr - px_r
            vy = pyr - py_r
            vz = pzr - pz_r
            ll = jnp.sqrt(vx * vx + vy * vy + vz * vz)
            linv = 1.0 / (ll + 1e-9)
            ux = vx * linv
            uy = vy * linv
            uz = vz * linv

            y1x = _C1 * ux
            y1y = _C1 * uy
            y1z = _C1 * uz
            sh = [None] * 16
            sh[0] = jnp.ones((BR, N_NODES), f32)
            sh[1] = y1x
            sh[2] = y1y
            sh[3] = y1z
            sh[4] = _S15 * ux * uy
            sh[5] = _S15 * uy * uz
            sh[6] = 0.5 * _S5 * (3.0 * uz * uz - 1.0)
            sh[7] = _S15 * ux * uz
            sh[8] = 0.5 * _S15 * (ux * ux - uy * uy)
            sh[9] = 0.25 * _S70 * uy * (3.0 * ux * ux - uy * uy)
            sh[10] = _S105 * ux * uy * uz
            sh[11] = 0.25 * _S42 * uy * (5.0 * uz * uz - 1.0)
            sh[12] = 0.5 * _S7 * uz * (5.0 * uz * uz - 3.0)
            sh[13] = 0.25 * _S42 * ux * (5.0 * uz * uz - 1.0)
            sh[14] = 0.5 * _S105 * uz * (ux * ux - uy * uy)
            sh[15] = 0.25 * _S70 * ux * (ux * ux - 3.0 * uy * uy)
            shw = sh[0][:, :, None] * wsh[0][None, None, :]
            for j in range(1, 16):
                shw = shw + sh[j][:, :, None] * wsh[j][None, None, :]

            theta = (jnp.pi / R_MAX) * ll
            s1 = jnp.sin(theta)
            c2 = 2.0 * jnp.cos(theta)
            f_prev = jnp.zeros((BR, N_NODES), f32)
            f_cur = s1
            z = br1[None, :, :] + (f_cur * linv)[:, :, None] * \
                wr1[0][None, None, :]
            for k in range(1, N_RADIAL):
                f_next = c2 * f_cur - f_prev
                f_prev, f_cur = f_cur, f_next
                z = z + (f_cur * linv)[:, :, None] * wr1[k][None, None, :]
            fcb = fc0_ref[pl.ds(r0, BR), :][:, :, None]
            h_rad = jax.nn.silu(z) * fcb
            rall = _mm(h_rad.reshape(BR * N_NODES, 64),
                       wr2).reshape(BR, N_NODES, 5 * F)
            r0c = rall[:, :, 0:F]
            r1c = rall[:, :, F:2 * F]
            r2c = rall[:, :, 2 * F:3 * F]
            r3c = rall[:, :, 3 * F:4 * F]
            r4c = rall[:, :, 4 * F:5 * F]

            hs_b = hsrc[None, :, :]
            yv = (y1x[:, :, None] * hvxs[None, :, :]
                  + y1y[:, :, None] * hvys[None, :, :]
                  + y1z[:, :, None] * hvzs[None, :, :])
            ms = r0c * hs_b + r2c * yv + r4c * shw
            aggs_ref[pl.ds(r0, BR), :] = jnp.sum(ms, axis=1)
            t1 = r1c * hs_b
            aggvx_ref[pl.ds(r0, BR), :] = jnp.sum(
                y1x[:, :, None] * t1 + r3c * hvxs[None, :, :], axis=1)
            aggvy_ref[pl.ds(r0, BR), :] = jnp.sum(
                y1y[:, :, None] * t1 + r3c * hvys[None, :, :], axis=1)
            aggvz_ref[pl.ds(r0, BR), :] = jnp.sum(
                y1z[:, :, None] * t1 + r3c * hvzs[None, :, :], axis=1)
            return 0

        jax.lax.fori_loop(0, NB, rb_body, 0)

        inv = 1.0 / AVG_NEI
        aggs = aggs_ref[...] * inv
        aggvx = aggvx_ref[...] * inv
        aggvy = aggvy_ref[...] * inv
        aggvz = aggvz_ref[...] * inv

        aggs2 = aggs * aggs
        h_s = (_mm(aggs, p1_ref[...]) + _mm(aggs2, p2_ref[...])
               + _mm(aggs2 * aggs, p3_ref[...]))
        gate = jax.nn.silu(_mm(h_s, wg_ref[...]))
        h_vx = _mm(aggvx, pv_ref[...]) * gate
        h_vy = _mm(aggvy, pv_ref[...]) * gate
        h_vz = _mm(aggvz, pv_ref[...]) * gate
        wvec = wvec_ref[...]
        px_c = px_c + jnp.sum(h_vx * wvec, axis=1)[:, None]
        py_c = py_c + jnp.sum(h_vy * wvec, axis=1)[:, None]
        pz_c = pz_c + jnp.sum(h_vz * wvec, axis=1)[:, None]
        px_r = jnp.transpose(px_c, (1, 0))
        py_r = jnp.transpose(py_c, (1, 0))
        pz_r = jnp.transpose(pz_c, (1, 0))

    out_ref[...] = jnp.concatenate(
        [px_c - px0_c, py_c - py0_c, pz_c - pz0_c,
         jnp.zeros((N_NODES, 1), f32)], axis=1)


@jax.jit
def _run(pos, emb, wemb, bemb, kvec, *layer_params):
    out = pl.pallas_call(
        _forward_body,
        out_shape=jax.ShapeDtypeStruct((N_NODES, 4), jnp.float32),
        scratch_shapes=[pltpu.VMEM((N_NODES, F), jnp.float32)] * 4
        + [pltpu.VMEM((N_NODES, N_NODES), jnp.float32),
           pltpu.VMEM((N_NODES, 4), jnp.float32)],
    )(pos, emb, wemb, bemb, kvec, *layer_params)
    return out[:, :3]


def kernel(positions, node_features, global_features, params):
    node_attrs = jax.nn.one_hot(node_features - 1, NUM_SPECIES)
    t = jnp.tile(global_features[None, :], (N_NODES, 1))
    emb = jnp.concatenate(
        [node_attrs, t,
         jnp.zeros((N_NODES, 3), jnp.float32)], axis=-1)
    wemb = jnp.concatenate(
        [params['W_emb'], jnp.zeros((3, F), jnp.float32)], axis=0)
    bemb = params['b_emb'][None, :]
    layer_params = []
    for p in params['layers']:
        layer_params += [
            p['Wr1'], p['br1'][None, :], p['Wr2'], p['Wsh'], p['Ws'],
            p['Wv'], p['P1'], p['P2'], p['P3'], p['Pv'], p['Wg'],
            p['wvec'][None, :],
        ]
    kvec = jnp.asarray(
        (np.arange(1, N_RADIAL + 1) * np.pi / R_MAX)[None, :],
        jnp.float32)
    return _run(positions.astype(jnp.float32), emb, wemb, bemb, kvec,
                *layer_params)


# speedup vs baseline: 1.1572x; 1.1572x over previous
import functools

import jax
import jax.numpy as jnp
import numpy as np
from jax.experimental import pallas as pl
from jax.experimental.pallas import tpu as pltpu

N_NODES = 256
F = 128
NUM_SPECIES = 5
TDIM = 32
R_MAX = 5.0
N_LAYERS = 2
AVG_NEI = 255.0
N_RADIAL = 8

BR = 16
NB = N_NODES // BR

_C1 = float(np.sqrt(3.0))
_S15 = float(np.sqrt(15.0))
_S5 = float(np.sqrt(5.0))
_S70 = float(np.sqrt(70.0))
_S105 = float(np.sqrt(105.0))
_S42 = float(np.sqrt(42.0))
_S7 = float(np.sqrt(7.0))


def _mm(a, b):
    return jax.lax.dot_general(a, b, (((1,), (0,)), ((), ())),
                               preferred_element_type=jnp.float32)


def _b16(x):
    return x.astype(jnp.bfloat16).astype(jnp.float32)


def _forward_body(pos_ref, emb_ref, wemb_ref, bemb_ref, kvec_ref, *rest):
    nparams = 12 * N_LAYERS
    layer_refs = rest[:nparams]
    out_ref = rest[nparams]
    (aggs_ref, aggvx_ref, aggvy_ref, aggvz_ref, fc0_ref,
     pcol_ref) = rest[nparams + 1:]

    f32 = jnp.float32

    px_c = pos_ref[:, 0:1]
    py_c = pos_ref[:, 1:2]
    pz_c = pos_ref[:, 2:3]
    px_r = jnp.transpose(px_c, (1, 0))
    py_r = jnp.transpose(py_c, (1, 0))
    pz_r = jnp.transpose(pz_c, (1, 0))
    px0_c, py0_c, pz0_c = px_c, py_c, pz_c

    vx0 = px_c - px_r
    vy0 = py_c - py_r
    vz0 = pz_c - pz_r
    l0 = jnp.sqrt(vx0 * vx0 + vy0 * vy0 + vz0 * vz0)
    fc = 0.5 * (jnp.cos((jnp.pi / R_MAX) * jnp.clip(l0, 0.0, R_MAX)) + 1.0)
    fc = fc * (l0 < R_MAX).astype(f32)
    ii = jax.lax.broadcasted_iota(jnp.int32, (N_NODES, N_NODES), 0)
    jj = jax.lax.broadcasted_iota(jnp.int32, (N_NODES, N_NODES), 1)
    fc0_ref[...] = jnp.where(ii == jj, 0.0, fc)

    h_s = _mm(emb_ref[...], wemb_ref[...]) + bemb_ref[...]
    h_vx = jnp.zeros((N_NODES, F), f32)
    h_vy = jnp.zeros((N_NODES, F), f32)
    h_vz = jnp.zeros((N_NODES, F), f32)

    del kvec_ref

    for l in range(N_LAYERS):
        (wr1_ref, br1_ref, wr2_ref, wsh_ref, ws_ref, wv_ref, p1_ref, p2_ref,
         p3_ref, pv_ref, wg_ref, wvec_ref) = layer_refs[12 * l:12 * (l + 1)]

        hsrc = _mm(h_s, ws_ref[...])
        hvxs = _mm(h_vx, wv_ref[...])
        hvys = _mm(h_vy, wv_ref[...])
        hvzs = _mm(h_vz, wv_ref[...])
        wr1 = _b16(wr1_ref[...])
        br1 = br1_ref[...]
        wr2 = wr2_ref[...]
        wsh = _b16(wsh_ref[...])
        pcol_ref[:, 0:1] = px_c
        pcol_ref[:, 1:2] = py_c
        pcol_ref[:, 2:3] = pz_c

        def rb_body(rb, _, px_r=px_r, py_r=py_r, pz_r=pz_r, hsrc=hsrc,
                    hvxs=hvxs, hvys=hvys, hvzs=hvzs, wr1=wr1, br1=br1,
                    wr2=wr2, wsh=wsh):
            r0 = rb * BR
            pxr = pcol_ref[pl.ds(r0, BR), 0:1]
            pyr = pcol_ref[pl.ds(r0, BR), 1:2]
            pzr = pcol_ref[pl.ds(r0, BR), 2:3]
            vx = pxr - px_r
            vy = pyr - py_r
            vz = pzr - pz_r
            ll = jnp.sqrt(vx * vx + vy * vy + vz * vz)
            linv = 1.0 / (ll + 1e-9)
            ux = vx * linv
            uy = vy * linv
            uz = vz * linv

            y1x = _C1 * ux
            y1y = _C1 * uy
            y1z = _C1 * uz
            sh = [None] * 16
            sh[0] = jnp.ones((BR, N_NODES), f32)
            sh[1] = y1x
            sh[2] = y1y
            sh[3] = y1z
            sh[4] = _S15 * ux * uy
            sh[5] = _S15 * uy * uz
            sh[6] = 0.5 * _S5 * (3.0 * uz * uz - 1.0)
            sh[7] = _S15 * ux * uz
            sh[8] = 0.5 * _S15 * (ux * ux - uy * uy)
            sh[9] = 0.25 * _S70 * uy * (3.0 * ux * ux - uy * uy)
            sh[10] = _S105 * ux * uy * uz
            sh[11] = 0.25 * _S42 * uy * (5.0 * uz * uz - 1.0)
            sh[12] = 0.5 * _S7 * uz * (5.0 * uz * uz - 3.0)
            sh[13] = 0.25 * _S42 * ux * (5.0 * uz * uz - 1.0)
            sh[14] = 0.5 * _S105 * uz * (ux * ux - uy * uy)
            sh[15] = 0.25 * _S70 * ux * (ux * ux - 3.0 * uy * uy)
            shw = _b16(sh[0])[:, :, None] * wsh[0][None, None, :]
            for j in range(1, 16):
                shw = shw + _b16(sh[j])[:, :, None] * wsh[j][None, None, :]

            theta = (jnp.pi / R_MAX) * ll
            s1 = jnp.sin(theta)
            c2 = 2.0 * jnp.cos(theta)
            f_prev = jnp.zeros((BR, N_NODES), f32)
            f_cur = s1
            z = _b16(f_cur * linv)[:, :, None] * wr1[0][None, None, :]
            for k in range(1, N_RADIAL):
                f_next = c2 * f_cur - f_prev
                f_prev, f_cur = f_cur, f_next
                z = z + _b16(f_cur * linv)[:, :, None] * \
                    wr1[k][None, None, :]
            h_rad = jax.nn.silu(z + br1[None, :, :])
            rall = _mm(h_rad.reshape(BR * N_NODES, 64),
                       wr2).reshape(BR, N_NODES, 5 * F)
            fcb = fc0_ref[pl.ds(r0, BR), :][:, :, None]
            r0c = rall[:, :, 0:F] * fcb
            r1c = rall[:, :, F:2 * F] * fcb
            r2c = rall[:, :, 2 * F:3 * F] * fcb
            r3c = rall[:, :, 3 * F:4 * F] * fcb
            r4c = rall[:, :, 4 * F:5 * F] * fcb

            hs_b = hsrc[None, :, :]
            yv = (y1x[:, :, None] * hvxs[None, :, :]
                  + y1y[:, :, None] * hvys[None, :, :]
                  + y1z[:, :, None] * hvzs[None, :, :])
            ms = r0c * hs_b + r2c * yv + r4c * shw
            aggs_ref[pl.ds(r0, BR), :] = jnp.sum(ms, axis=1)
            t1 = r1c * hs_b
            aggvx_ref[pl.ds(r0, BR), :] = jnp.sum(
                y1x[:, :, None] * t1 + r3c * hvxs[None, :, :], axis=1)
            aggvy_ref[pl.ds(r0, BR), :] = jnp.sum(
                y1y[:, :, None] * t1 + r3c * hvys[None, :, :], axis=1)
            aggvz_ref[pl.ds(r0, BR), :] = jnp.sum(
                y1z[:, :, None] * t1 + r3c * hvzs[None, :, :], axis=1)
            return 0

        jax.lax.fori_loop(0, NB, rb_body, 0)

        inv = 1.0 / AVG_NEI
        aggs = aggs_ref[...] * inv
        aggvx = aggvx_ref[...] * inv
        aggvy = aggvy_ref[...] * inv
        aggvz = aggvz_ref[...] * inv

        aggs2 = aggs * aggs
        h_s = (_mm(aggs, p1_ref[...]) + _mm(aggs2, p2_ref[...])
               + _mm(aggs2 * aggs, p3_ref[...]))
        gate = jax.nn.silu(_mm(h_s, wg_ref[...]))
        h_vx = _mm(aggvx, pv_ref[...]) * gate
        h_vy = _mm(aggvy, pv_ref[...]) * gate
        h_vz = _mm(aggvz, pv_ref[...]) * gate
        wvec = _b16(wvec_ref[...])
        px_c = px_c + jnp.sum(_b16(h_vx) * wvec, axis=1)[:, None]
        py_c = py_c + jnp.sum(_b16(h_vy) * wvec, axis=1)[:, None]
        pz_c = pz_c + jnp.sum(_b16(h_vz) * wvec, axis=1)[:, None]
        px_r = jnp.transpose(px_c, (1, 0))
        py_r = jnp.transpose(py_c, (1, 0))
        pz_r = jnp.transpose(pz_c, (1, 0))

    out_ref[...] = jnp.concatenate(
        [px_c - px0_c, py_c - py0_c, pz_c - pz0_c,
         jnp.zeros((N_NODES, 1), f32)], axis=1)


@jax.jit
def _run(pos, emb, wemb, bemb, kvec, *layer_params):
    out = pl.pallas_call(
        _forward_body,
        out_shape=jax.ShapeDtypeStruct((N_NODES, 4), jnp.float32),
        scratch_shapes=[pltpu.VMEM((N_NODES, F), jnp.float32)] * 4
        + [pltpu.VMEM((N_NODES, N_NODES), jnp.float32),
           pltpu.VMEM((N_NODES, 4), jnp.float32)],
    )(pos, emb, wemb, bemb, kvec, *layer_params)
    return out[:, :3]


def kernel(positions, node_features, global_features, params):
    node_attrs = jax.nn.one_hot(node_features - 1, NUM_SPECIES)
    t = jnp.tile(global_features[None, :], (N_NODES, 1))
    emb = jnp.concatenate(
        [node_attrs, t,
         jnp.zeros((N_NODES, 3), jnp.float32)], axis=-1)
    wemb = jnp.concatenate(
        [params['W_emb'], jnp.zeros((3, F), jnp.float32)], axis=0)
    bemb = params['b_emb'][None, :]
    layer_params = []
    for p in params['layers']:
        layer_params += [
            p['Wr1'], p['br1'][None, :], p['Wr2'], p['Wsh'], p['Ws'],
            p['Wv'], p['P1'], p['P2'], p['P3'], p['Pv'], p['Wg'],
            p['wvec'][None, :],
        ]
    kvec = jnp.asarray(
        (np.arange(1, N_RADIAL + 1) * np.pi / R_MAX)[None, :],
        jnp.float32)
    return _run(positions.astype(jnp.float32), emb, wemb, bemb, kvec,
                *layer_params)


# speedup vs baseline: 4.1268x; 3.5663x over previous
import functools

import jax
import jax.numpy as jnp
import numpy as np
from jax.experimental import pallas as pl
from jax.experimental.pallas import tpu as pltpu

N_NODES = 256
F = 128
NUM_SPECIES = 5
TDIM = 32
R_MAX = 5.0
N_LAYERS = 2
AVG_NEI = 255.0
N_RADIAL = 8

BR = 16
NB = N_NODES // BR

_C1 = float(np.sqrt(3.0))
_S15 = float(np.sqrt(15.0))
_S5 = float(np.sqrt(5.0))
_S70 = float(np.sqrt(70.0))
_S105 = float(np.sqrt(105.0))
_S42 = float(np.sqrt(42.0))
_S7 = float(np.sqrt(7.0))


def _mm(a, b):
    return jax.lax.dot_general(a, b, (((1,), (0,)), ((), ())),
                               preferred_element_type=jnp.float32)


def _b16(x):
    return x.astype(jnp.bfloat16).astype(jnp.float32)


def _forward_body(pos_ref, emb_ref, wemb_ref, bemb_ref, kvec_ref, *rest):
    nparams = 12 * N_LAYERS
    layer_refs = rest[:nparams]
    out_ref = rest[nparams]
    (aggs_ref, aggvx_ref, aggvy_ref, aggvz_ref, fc0_ref,
     pcol_ref) = rest[nparams + 1:]

    f32 = jnp.float32

    px_c = pos_ref[:, 0:1]
    py_c = pos_ref[:, 1:2]
    pz_c = pos_ref[:, 2:3]
    px_r = jnp.transpose(px_c, (1, 0))
    py_r = jnp.transpose(py_c, (1, 0))
    pz_r = jnp.transpose(pz_c, (1, 0))
    px0_c, py0_c, pz0_c = px_c, py_c, pz_c

    vx0 = px_c - px_r
    vy0 = py_c - py_r
    vz0 = pz_c - pz_r
    l0 = jnp.sqrt(vx0 * vx0 + vy0 * vy0 + vz0 * vz0)
    fc = 0.5 * (jnp.cos((jnp.pi / R_MAX) * jnp.clip(l0, 0.0, R_MAX)) + 1.0)
    fc = fc * (l0 < R_MAX).astype(f32)
    ii = jax.lax.broadcasted_iota(jnp.int32, (N_NODES, N_NODES), 0)
    jj = jax.lax.broadcasted_iota(jnp.int32, (N_NODES, N_NODES), 1)
    fc0_ref[...] = jnp.where(ii == jj, 0.0, fc)

    h_s = _mm(emb_ref[...], wemb_ref[...]) + bemb_ref[...]
    h_vx = jnp.zeros((N_NODES, F), f32)
    h_vy = jnp.zeros((N_NODES, F), f32)
    h_vz = jnp.zeros((N_NODES, F), f32)

    del kvec_ref

    for l in range(N_LAYERS):
        (wr1_ref, br1_ref, wr2_ref, wsh_ref, ws_ref, wv_ref, p1_ref, p2_ref,
         p3_ref, pv_ref, wg_ref, wvec_ref) = layer_refs[12 * l:12 * (l + 1)]

        hsrc = _mm(h_s, ws_ref[...])
        hvxs = _mm(h_vx, wv_ref[...])
        hvys = _mm(h_vy, wv_ref[...])
        hvzs = _mm(h_vz, wv_ref[...])
        wr1 = wr1_ref[...]
        br1 = br1_ref[...]
        wr2 = wr2_ref[...]
        wsh = wsh_ref[...]
        pcol_ref[:, 0:1] = px_c
        pcol_ref[:, 1:2] = py_c
        pcol_ref[:, 2:3] = pz_c

        def rb_body(rb, _, px_r=px_r, py_r=py_r, pz_r=pz_r, hsrc=hsrc,
                    hvxs=hvxs, hvys=hvys, hvzs=hvzs, wr1=wr1, br1=br1,
                    wr2=wr2, wsh=wsh):
            r0 = rb * BR
            pxr = pcol_ref[pl.ds(r0, BR), 0:1]
            pyr = pcol_ref[pl.ds(r0, BR), 1:2]
            pzr = pcol_ref[pl.ds(r0, BR), 2:3]
            vx = pxr - px_r
            vy = pyr - py_r
            vz = pzr - pz_r
            ll = jnp.sqrt(vx * vx + vy * vy + vz * vz)
            linv = 1.0 / (ll + 1e-9)
            ux = vx * linv
            uy = vy * linv
            uz = vz * linv

            y1x = _C1 * ux
            y1y = _C1 * uy
            y1z = _C1 * uz
            sh = [None] * 16
            sh[0] = jnp.ones((BR, N_NODES), f32)
            sh[1] = y1x
            sh[2] = y1y
            sh[3] = y1z
            sh[4] = _S15 * ux * uy
            sh[5] = _S15 * uy * uz
            sh[6] = 0.5 * _S5 * (3.0 * uz * uz - 1.0)
            sh[7] = _S15 * ux * uz
            sh[8] = 0.5 * _S15 * (ux * ux - uy * uy)
            sh[9] = 0.25 * _S70 * uy * (3.0 * ux * ux - uy * uy)
            sh[10] = _S105 * ux * uy * uz
            sh[11] = 0.25 * _S42 * uy * (5.0 * uz * uz - 1.0)
            sh[12] = 0.5 * _S7 * uz * (5.0 * uz * uz - 3.0)
            sh[13] = 0.25 * _S42 * ux * (5.0 * uz * uz - 1.0)
            sh[14] = 0.5 * _S105 * uz * (ux * ux - uy * uy)
            sh[15] = 0.25 * _S70 * ux * (ux * ux - 3.0 * uy * uy)
            shst = jnp.concatenate([s[:, None, :] for s in sh], axis=1)
            shw = jax.lax.dot_general(
                shst, wsh, (((1,), (0,)), ((), ())),
                preferred_element_type=f32)

            theta = (jnp.pi / R_MAX) * ll
            s1 = jnp.sin(theta)
            c2 = 2.0 * jnp.cos(theta)
            frows = [s1 * linv]
            f_prev, f_cur = jnp.zeros((BR, N_NODES), f32), s1
            for k in range(1, N_RADIAL):
                f_next = c2 * f_cur - f_prev
                f_prev, f_cur = f_cur, f_next
                frows.append(f_cur * linv)
            featst = jnp.concatenate([f[:, None, :] for f in frows], axis=1)
            z = jax.lax.dot_general(
                featst, wr1, (((1,), (0,)), ((), ())),
                preferred_element_type=f32)
            h_rad = jax.nn.silu(z + br1[None, :, :])
            rall = _mm(h_rad.reshape(BR * N_NODES, 64),
                       wr2).reshape(BR, N_NODES, 5 * F)
            fcb = fc0_ref[pl.ds(r0, BR), :][:, :, None]
            r0c = rall[:, :, 0:F] * fcb
            r1c = rall[:, :, F:2 * F] * fcb
            r2c = rall[:, :, 2 * F:3 * F] * fcb
            r3c = rall[:, :, 3 * F:4 * F] * fcb
            r4c = rall[:, :, 4 * F:5 * F] * fcb

            hs_b = hsrc[None, :, :]
            yv = (y1x[:, :, None] * hvxs[None, :, :]
                  + y1y[:, :, None] * hvys[None, :, :]
                  + y1z[:, :, None] * hvzs[None, :, :])
            ms = r0c * hs_b + r2c * yv + r4c * shw
            aggs_ref[pl.ds(r0, BR), :] = jnp.sum(ms, axis=1)
            t1 = r1c * hs_b
            aggvx_ref[pl.ds(r0, BR), :] = jnp.sum(
                y1x[:, :, None] * t1 + r3c * hvxs[None, :, :], axis=1)
            aggvy_ref[pl.ds(r0, BR), :] = jnp.sum(
                y1y[:, :, None] * t1 + r3c * hvys[None, :, :], axis=1)
            aggvz_ref[pl.ds(r0, BR), :] = jnp.sum(
                y1z[:, :, None] * t1 + r3c * hvzs[None, :, :], axis=1)
            return 0

        jax.lax.fori_loop(0, NB, rb_body, 0)

        inv = 1.0 / AVG_NEI
        aggs = aggs_ref[...] * inv
        aggvx = aggvx_ref[...] * inv
        aggvy = aggvy_ref[...] * inv
        aggvz = aggvz_ref[...] * inv

        aggs2 = aggs * aggs
        h_s = (_mm(aggs, p1_ref[...]) + _mm(aggs2, p2_ref[...])
               + _mm(aggs2 * aggs, p3_ref[...]))
        gate = jax.nn.silu(_mm(h_s, wg_ref[...]))
        h_vx = _mm(aggvx, pv_ref[...]) * gate
        h_vy = _mm(aggvy, pv_ref[...]) * gate
        h_vz = _mm(aggvz, pv_ref[...]) * gate
        wvec = _b16(wvec_ref[...])
        px_c = px_c + jnp.sum(_b16(h_vx) * wvec, axis=1)[:, None]
        py_c = py_c + jnp.sum(_b16(h_vy) * wvec, axis=1)[:, None]
        pz_c = pz_c + jnp.sum(_b16(h_vz) * wvec, axis=1)[:, None]
        px_r = jnp.transpose(px_c, (1, 0))
        py_r = jnp.transpose(py_c, (1, 0))
        pz_r = jnp.transpose(pz_c, (1, 0))

    out_ref[...] = jnp.concatenate(
        [px_c - px0_c, py_c - py0_c, pz_c - pz0_c,
         jnp.zeros((N_NODES, 1), f32)], axis=1)


@jax.jit
def _run(pos, emb, wemb, bemb, kvec, *layer_params):
    out = pl.pallas_call(
        _forward_body,
        out_shape=jax.ShapeDtypeStruct((N_NODES, 4), jnp.float32),
        scratch_shapes=[pltpu.VMEM((N_NODES, F), jnp.float32)] * 4
        + [pltpu.VMEM((N_NODES, N_NODES), jnp.float32),
           pltpu.VMEM((N_NODES, 4), jnp.float32)],
    )(pos, emb, wemb, bemb, kvec, *layer_params)
    return out[:, :3]


def kernel(positions, node_features, global_features, params):
    node_attrs = jax.nn.one_hot(node_features - 1, NUM_SPECIES)
    t = jnp.tile(global_features[None, :], (N_NODES, 1))
    emb = jnp.concatenate(
        [node_attrs, t,
         jnp.zeros((N_NODES, 3), jnp.float32)], axis=-1)
    wemb = jnp.concatenate(
        [params['W_emb'], jnp.zeros((3, F), jnp.float32)], axis=0)
    bemb = params['b_emb'][None, :]
    layer_params = []
    for p in params['layers']:
        layer_params += [
            p['Wr1'], p['br1'][None, :], p['Wr2'], p['Wsh'], p['Ws'],
            p['Wv'], p['P1'], p['P2'], p['P3'], p['Pv'], p['Wg'],
            p['wvec'][None, :],
        ]
    kvec = jnp.asarray(
        (np.arange(1, N_RADIAL + 1) * np.pi / R_MAX)[None, :],
        jnp.float32)
    return _run(positions.astype(jnp.float32), emb, wemb, bemb, kvec,
                *layer_params)


# speedup vs baseline: 4.1962x; 1.0168x over previous
import functools

import jax
import jax.numpy as jnp
import numpy as np
from jax.experimental import pallas as pl
from jax.experimental.pallas import tpu as pltpu

N_NODES = 256
F = 128
NUM_SPECIES = 5
TDIM = 32
R_MAX = 5.0
N_LAYERS = 2
AVG_NEI = 255.0
N_RADIAL = 8

BR = 32
NB = N_NODES // BR

_C1 = float(np.sqrt(3.0))
_S15 = float(np.sqrt(15.0))
_S5 = float(np.sqrt(5.0))
_S70 = float(np.sqrt(70.0))
_S105 = float(np.sqrt(105.0))
_S42 = float(np.sqrt(42.0))
_S7 = float(np.sqrt(7.0))


def _mm(a, b):
    return jax.lax.dot_general(a, b, (((1,), (0,)), ((), ())),
                               preferred_element_type=jnp.float32)


def _b16(x):
    return x.astype(jnp.bfloat16).astype(jnp.float32)


def _forward_body(pos_ref, emb_ref, wemb_ref, bemb_ref, kvec_ref, *rest):
    nparams = 12 * N_LAYERS
    layer_refs = rest[:nparams]
    out_ref = rest[nparams]
    (aggs_ref, aggvx_ref, aggvy_ref, aggvz_ref, fc0_ref,
     pcol_ref) = rest[nparams + 1:]

    f32 = jnp.float32

    px_c = pos_ref[:, 0:1]
    py_c = pos_ref[:, 1:2]
    pz_c = pos_ref[:, 2:3]
    px_r = jnp.transpose(px_c, (1, 0))
    py_r = jnp.transpose(py_c, (1, 0))
    pz_r = jnp.transpose(pz_c, (1, 0))
    px0_c, py0_c, pz0_c = px_c, py_c, pz_c

    vx0 = px_c - px_r
    vy0 = py_c - py_r
    vz0 = pz_c - pz_r
    l0 = jnp.sqrt(vx0 * vx0 + vy0 * vy0 + vz0 * vz0)
    fc = 0.5 * (jnp.cos((jnp.pi / R_MAX) * jnp.clip(l0, 0.0, R_MAX)) + 1.0)
    fc = fc * (l0 < R_MAX).astype(f32)
    ii = jax.lax.broadcasted_iota(jnp.int32, (N_NODES, N_NODES), 0)
    jj = jax.lax.broadcasted_iota(jnp.int32, (N_NODES, N_NODES), 1)
    fc0_ref[...] = jnp.where(ii == jj, 0.0, fc)

    h_s = _mm(emb_ref[...], wemb_ref[...]) + bemb_ref[...]
    h_vx = jnp.zeros((N_NODES, F), f32)
    h_vy = jnp.zeros((N_NODES, F), f32)
    h_vz = jnp.zeros((N_NODES, F), f32)

    del kvec_ref

    for l in range(N_LAYERS):
        (wr1_ref, br1_ref, wr2_ref, wsh_ref, ws_ref, wv_ref, p1_ref, p2_ref,
         p3_ref, pv_ref, wg_ref, wvec_ref) = layer_refs[12 * l:12 * (l + 1)]

        hsrc = _mm(h_s, ws_ref[...])
        hvxs = _mm(h_vx, wv_ref[...])
        hvys = _mm(h_vy, wv_ref[...])
        hvzs = _mm(h_vz, wv_ref[...])
        wr1 = wr1_ref[...]
        br1 = br1_ref[...]
        wr2 = wr2_ref[...]
        wsh = wsh_ref[...]
        pcol_ref[:, 0:1] = px_c
        pcol_ref[:, 1:2] = py_c
        pcol_ref[:, 2:3] = pz_c

        def rb_body(rb, _, px_r=px_r, py_r=py_r, pz_r=pz_r, hsrc=hsrc,
                    hvxs=hvxs, hvys=hvys, hvzs=hvzs, wr1=wr1, br1=br1,
                    wr2=wr2, wsh=wsh):
            r0 = rb * BR
            pxr = pcol_ref[pl.ds(r0, BR), 0:1]
            pyr = pcol_ref[pl.ds(r0, BR), 1:2]
            pzr = pcol_ref[pl.ds(r0, BR), 2:3]
            vx = pxr - px_r
            vy = pyr - py_r
            vz = pzr - pz_r
            ll = jnp.sqrt(vx * vx + vy * vy + vz * vz)
            linv = 1.0 / (ll + 1e-9)
            ux = vx * linv
            uy = vy * linv
            uz = vz * linv

            y1x = _C1 * ux
            y1y = _C1 * uy
            y1z = _C1 * uz
            sh = [None] * 16
            sh[0] = jnp.ones((BR, N_NODES), f32)
            sh[1] = y1x
            sh[2] = y1y
            sh[3] = y1z
            sh[4] = _S15 * ux * uy
            sh[5] = _S15 * uy * uz
            sh[6] = 0.5 * _S5 * (3.0 * uz * uz - 1.0)
            sh[7] = _S15 * ux * uz
            sh[8] = 0.5 * _S15 * (ux * ux - uy * uy)
            sh[9] = 0.25 * _S70 * uy * (3.0 * ux * ux - uy * uy)
            sh[10] = _S105 * ux * uy * uz
            sh[11] = 0.25 * _S42 * uy * (5.0 * uz * uz - 1.0)
            sh[12] = 0.5 * _S7 * uz * (5.0 * uz * uz - 3.0)
            sh[13] = 0.25 * _S42 * ux * (5.0 * uz * uz - 1.0)
            sh[14] = 0.5 * _S105 * uz * (ux * ux - uy * uy)
            sh[15] = 0.25 * _S70 * ux * (ux * ux - 3.0 * uy * uy)
            shst = jnp.concatenate([s[:, None, :] for s in sh], axis=1)
            shw = jax.lax.dot_general(
                shst, wsh, (((1,), (0,)), ((), ())),
                preferred_element_type=f32)

            theta = (jnp.pi / R_MAX) * ll
            s1 = jnp.sin(theta)
            c2 = 2.0 * jnp.cos(theta)
            frows = [s1 * linv]
            f_prev, f_cur = jnp.zeros((BR, N_NODES), f32), s1
            for k in range(1, N_RADIAL):
                f_next = c2 * f_cur - f_prev
                f_prev, f_cur = f_cur, f_next
                frows.append(f_cur * linv)
            featst = jnp.concatenate([f[:, None, :] for f in frows], axis=1)
            z = jax.lax.dot_general(
                featst, wr1, (((1,), (0,)), ((), ())),
                preferred_element_type=f32)
            h_rad = jax.nn.silu(z + br1[None, :, :])
            rall = _mm(h_rad.reshape(BR * N_NODES, 64),
                       wr2).reshape(BR, N_NODES, 5 * F)
            fcb = fc0_ref[pl.ds(r0, BR), :][:, :, None]
            r0c = rall[:, :, 0:F] * fcb
            r1c = rall[:, :, F:2 * F] * fcb
            r2c = rall[:, :, 2 * F:3 * F] * fcb
            r3c = rall[:, :, 3 * F:4 * F] * fcb
            r4c = rall[:, :, 4 * F:5 * F] * fcb

            hs_b = hsrc[None, :, :]
            yv = (y1x[:, :, None] * hvxs[None, :, :]
                  + y1y[:, :, None] * hvys[None, :, :]
                  + y1z[:, :, None] * hvzs[None, :, :])
            ms = r0c * hs_b + r2c * yv + r4c * shw
            aggs_ref[pl.ds(r0, BR), :] = jnp.sum(ms, axis=1)
            t1 = r1c * hs_b
            aggvx_ref[pl.ds(r0, BR), :] = jnp.sum(
                y1x[:, :, None] * t1 + r3c * hvxs[None, :, :], axis=1)
            aggvy_ref[pl.ds(r0, BR), :] = jnp.sum(
                y1y[:, :, None] * t1 + r3c * hvys[None, :, :], axis=1)
            aggvz_ref[pl.ds(r0, BR), :] = jnp.sum(
                y1z[:, :, None] * t1 + r3c * hvzs[None, :, :], axis=1)
            return 0

        jax.lax.fori_loop(0, NB, rb_body, 0)

        inv = 1.0 / AVG_NEI
        aggs = aggs_ref[...] * inv
        aggvx = aggvx_ref[...] * inv
        aggvy = aggvy_ref[...] * inv
        aggvz = aggvz_ref[...] * inv

        aggs2 = aggs * aggs
        h_s = (_mm(aggs, p1_ref[...]) + _mm(aggs2, p2_ref[...])
               + _mm(aggs2 * aggs, p3_ref[...]))
        gate = jax.nn.silu(_mm(h_s, wg_ref[...]))
        h_vx = _mm(aggvx, pv_ref[...]) * gate
        h_vy = _mm(aggvy, pv_ref[...]) * gate
        h_vz = _mm(aggvz, pv_ref[...]) * gate
        wvec = _b16(wvec_ref[...])
        px_c = px_c + jnp.sum(_b16(h_vx) * wvec, axis=1)[:, None]
        py_c = py_c + jnp.sum(_b16(h_vy) * wvec, axis=1)[:, None]
        pz_c = pz_c + jnp.sum(_b16(h_vz) * wvec, axis=1)[:, None]
        px_r = jnp.transpose(px_c, (1, 0))
        py_r = jnp.transpose(py_c, (1, 0))
        pz_r = jnp.transpose(pz_c, (1, 0))

    out_ref[...] = jnp.concatenate(
        [px_c - px0_c, py_c - py0_c, pz_c - pz0_c,
         jnp.zeros((N_NODES, 1), f32)], axis=1)


@jax.jit
def _run(pos, emb, wemb, bemb, kvec, *layer_params):
    out = pl.pallas_call(
        _forward_body,
        out_shape=jax.ShapeDtypeStruct((N_NODES, 4), jnp.float32),
        scratch_shapes=[pltpu.VMEM((N_NODES, F), jnp.float32)] * 4
        + [pltpu.VMEM((N_NODES, N_NODES), jnp.float32),
           pltpu.VMEM((N_NODES, 4), jnp.float32)],
    )(pos, emb, wemb, bemb, kvec, *layer_params)
    return out[:, :3]


def kernel(positions, node_features, global_features, params):
    node_attrs = jax.nn.one_hot(node_features - 1, NUM_SPECIES)
    t = jnp.tile(global_features[None, :], (N_NODES, 1))
    emb = jnp.concatenate(
        [node_attrs, t,
         jnp.zeros((N_NODES, 3), jnp.float32)], axis=-1)
    wemb = jnp.concatenate(
        [params['W_emb'], jnp.zeros((3, F), jnp.float32)], axis=0)
    bemb = params['b_emb'][None, :]
    layer_params = []
    for p in params['layers']:
        layer_params += [
            p['Wr1'], p['br1'][None, :], p['Wr2'], p['Wsh'], p['Ws'],
            p['Wv'], p['P1'], p['P2'], p['P3'], p['Pv'], p['Wg'],
            p['wvec'][None, :],
        ]
    kvec = jnp.asarray(
        (np.arange(1, N_RADIAL + 1) * np.pi / R_MAX)[None, :],
        jnp.float32)
    return _run(positions.astype(jnp.float32), emb, wemb, bemb, kvec,
                *layer_params)


# speedup vs baseline: 4.6602x; 1.1106x over previous
import functools

import jax
import jax.numpy as jnp
import numpy as np
from jax.experimental import pallas as pl
from jax.experimental.pallas import tpu as pltpu

N_NODES = 256
F = 128
NUM_SPECIES = 5
TDIM = 32
R_MAX = 5.0
N_LAYERS = 2
AVG_NEI = 255.0
N_RADIAL = 8

BR = 32
NB = N_NODES // BR

_C1 = float(np.sqrt(3.0))
_S15 = float(np.sqrt(15.0))
_S5 = float(np.sqrt(5.0))
_S70 = float(np.sqrt(70.0))
_S105 = float(np.sqrt(105.0))
_S42 = float(np.sqrt(42.0))
_S7 = float(np.sqrt(7.0))


def _mm(a, b):
    return jax.lax.dot_general(a, b, (((1,), (0,)), ((), ())),
                               preferred_element_type=jnp.float32)


def _b16(x):
    return x.astype(jnp.bfloat16).astype(jnp.float32)


def _forward_body(pos_ref, emb_ref, wemb_ref, bemb_ref, kvec_ref, *rest):
    nparams = 12 * N_LAYERS
    layer_refs = rest[:nparams]
    out_ref = rest[nparams]
    (aggs_ref, aggvx_ref, aggvy_ref, aggvz_ref, fc0_ref,
     pcol_ref) = rest[nparams + 1:]

    f32 = jnp.float32

    px_c = pos_ref[:, 0:1]
    py_c = pos_ref[:, 1:2]
    pz_c = pos_ref[:, 2:3]
    px_r = jnp.transpose(px_c, (1, 0))
    py_r = jnp.transpose(py_c, (1, 0))
    pz_r = jnp.transpose(pz_c, (1, 0))
    px0_c, py0_c, pz0_c = px_c, py_c, pz_c

    vx0 = px_c - px_r
    vy0 = py_c - py_r
    vz0 = pz_c - pz_r
    l0 = jnp.sqrt(vx0 * vx0 + vy0 * vy0 + vz0 * vz0)
    fc = 0.5 * (jnp.cos((jnp.pi / R_MAX) * jnp.clip(l0, 0.0, R_MAX)) + 1.0)
    fc = fc * (l0 < R_MAX).astype(f32)
    ii = jax.lax.broadcasted_iota(jnp.int32, (N_NODES, N_NODES), 0)
    jj = jax.lax.broadcasted_iota(jnp.int32, (N_NODES, N_NODES), 1)
    fc0_ref[...] = jnp.where(ii == jj, 0.0, fc)

    h_s = _mm(emb_ref[...], wemb_ref[...]) + bemb_ref[...]
    h_vx = jnp.zeros((N_NODES, F), f32)
    h_vy = jnp.zeros((N_NODES, F), f32)
    h_vz = jnp.zeros((N_NODES, F), f32)

    del kvec_ref

    for l in range(N_LAYERS):
        (wr1_ref, br1_ref, wr2_ref, wsh_ref, ws_ref, wv_ref, p1_ref, p2_ref,
         p3_ref, pv_ref, wg_ref, wvec_ref) = layer_refs[12 * l:12 * (l + 1)]

        hsrc = _mm(h_s, ws_ref[...])
        hvxs = _mm(h_vx, wv_ref[...])
        hvys = _mm(h_vy, wv_ref[...])
        hvzs = _mm(h_vz, wv_ref[...])
        wr1 = wr1_ref[...]
        br1 = br1_ref[...]
        wr2 = wr2_ref[...]
        wsh = wsh_ref[...]
        pcol_ref[:, 0:1] = px_c
        pcol_ref[:, 1:2] = py_c
        pcol_ref[:, 2:3] = pz_c

        def rb_body(rb, _, px_r=px_r, py_r=py_r, pz_r=pz_r, hsrc=hsrc,
                    hvxs=hvxs, hvys=hvys, hvzs=hvzs, wr1=wr1, br1=br1,
                    wr2=wr2, wsh=wsh):
            r0 = rb * BR
            pxr = pcol_ref[pl.ds(r0, BR), 0:1]
            pyr = pcol_ref[pl.ds(r0, BR), 1:2]
            pzr = pcol_ref[pl.ds(r0, BR), 2:3]
            vx = pxr - px_r
            vy = pyr - py_r
            vz = pzr - pz_r
            ll = jnp.sqrt(vx * vx + vy * vy + vz * vz)
            linv = 1.0 / (ll + 1e-9)
            ux = vx * linv
            uy = vy * linv
            uz = vz * linv

            y1x = _C1 * ux
            y1y = _C1 * uy
            y1z = _C1 * uz
            sh = [None] * 16
            sh[0] = jnp.ones((BR, N_NODES), f32)
            sh[1] = y1x
            sh[2] = y1y
            sh[3] = y1z
            sh[4] = _S15 * ux * uy
            sh[5] = _S15 * uy * uz
            sh[6] = 0.5 * _S5 * (3.0 * uz * uz - 1.0)
            sh[7] = _S15 * ux * uz
            sh[8] = 0.5 * _S15 * (ux * ux - uy * uy)
            sh[9] = 0.25 * _S70 * uy * (3.0 * ux * ux - uy * uy)
            sh[10] = _S105 * ux * uy * uz
            sh[11] = 0.25 * _S42 * uy * (5.0 * uz * uz - 1.0)
            sh[12] = 0.5 * _S7 * uz * (5.0 * uz * uz - 3.0)
            sh[13] = 0.25 * _S42 * ux * (5.0 * uz * uz - 1.0)
            sh[14] = 0.5 * _S105 * uz * (ux * ux - uy * uy)
            sh[15] = 0.25 * _S70 * ux * (ux * ux - 3.0 * uy * uy)
            shst = jnp.concatenate([s[:, None, :] for s in sh], axis=1)
            shw = jax.lax.dot_general(
                shst, wsh, (((1,), (0,)), ((), ())),
                preferred_element_type=f32)

            theta = (jnp.pi / R_MAX) * ll
            s1 = jnp.sin(theta)
            c2 = 2.0 * jnp.cos(theta)
            frows = [s1 * linv]
            f_prev, f_cur = jnp.zeros((BR, N_NODES), f32), s1
            for k in range(1, N_RADIAL):
                f_next = c2 * f_cur - f_prev
                f_prev, f_cur = f_cur, f_next
                frows.append(f_cur * linv)
            featst = jnp.concatenate([f[:, None, :] for f in frows], axis=1)
            z = jax.lax.dot_general(
                featst, wr1, (((1,), (0,)), ((), ())),
                preferred_element_type=f32)
            h_rad = jax.nn.silu(z + br1[None, :, :])
            rall = _mm(h_rad.reshape(BR * N_NODES, 64),
                       wr2).reshape(BR, N_NODES, 5 * F)
            fcb = fc0_ref[pl.ds(r0, BR), :][:, :, None]
            r0c = rall[:, :, 0:F] * fcb
            r1c = rall[:, :, F:2 * F] * fcb
            r2c = rall[:, :, 2 * F:3 * F] * fcb
            r3c = rall[:, :, 3 * F:4 * F] * fcb
            r4c = rall[:, :, 4 * F:5 * F] * fcb

            hs_b = hsrc[None, :, :]
            yv = (y1x[:, :, None] * hvxs[None, :, :]
                  + y1y[:, :, None] * hvys[None, :, :]
                  + y1z[:, :, None] * hvzs[None, :, :])
            ms = r0c * hs_b + r2c * yv + r4c * shw
            aggs_ref[pl.ds(r0, BR), :] = jnp.sum(ms, axis=1)
            t1 = r1c * hs_b
            y1b = jnp.concatenate(
                [y1x[:, None, :], y1y[:, None, :], y1z[:, None, :]], axis=1)
            aggv_y1 = jax.lax.dot_general(
                y1b, t1, (((2,), (1,)), ((0,), (0,))),
                preferred_element_type=f32)
            aggvx_ref[pl.ds(r0, BR), :] = aggv_y1[:, 0, :] + jnp.sum(
                r3c * hvxs[None, :, :], axis=1)
            aggvy_ref[pl.ds(r0, BR), :] = aggv_y1[:, 1, :] + jnp.sum(
                r3c * hvys[None, :, :], axis=1)
            aggvz_ref[pl.ds(r0, BR), :] = aggv_y1[:, 2, :] + jnp.sum(
                r3c * hvzs[None, :, :], axis=1)
            return 0

        jax.lax.fori_loop(0, NB, rb_body, 0)

        inv = 1.0 / AVG_NEI
        aggs = aggs_ref[...] * inv
        aggvx = aggvx_ref[...] * inv
        aggvy = aggvy_ref[...] * inv
        aggvz = aggvz_ref[...] * inv

        aggs2 = aggs * aggs
        h_s = (_mm(aggs, p1_ref[...]) + _mm(aggs2, p2_ref[...])
               + _mm(aggs2 * aggs, p3_ref[...]))
        gate = jax.nn.silu(_mm(h_s, wg_ref[...]))
        h_vx = _mm(aggvx, pv_ref[...]) * gate
        h_vy = _mm(aggvy, pv_ref[...]) * gate
        h_vz = _mm(aggvz, pv_ref[...]) * gate
        wvec = _b16(wvec_ref[...])
        px_c = px_c + jnp.sum(_b16(h_vx) * wvec, axis=1)[:, None]
        py_c = py_c + jnp.sum(_b16(h_vy) * wvec, axis=1)[:, None]
        pz_c = pz_c + jnp.sum(_b16(h_vz) * wvec, axis=1)[:, None]
        px_r = jnp.transpose(px_c, (1, 0))
        py_r = jnp.transpose(py_c, (1, 0))
        pz_r = jnp.transpose(pz_c, (1, 0))

    out_ref[...] = jnp.concatenate(
        [px_c - px0_c, py_c - py0_c, pz_c - pz0_c,
         jnp.zeros((N_NODES, 1), f32)], axis=1)


@jax.jit
def _run(pos, emb, wemb, bemb, kvec, *layer_params):
    out = pl.pallas_call(
        _forward_body,
        out_shape=jax.ShapeDtypeStruct((N_NODES, 4), jnp.float32),
        scratch_shapes=[pltpu.VMEM((N_NODES, F), jnp.float32)] * 4
        + [pltpu.VMEM((N_NODES, N_NODES), jnp.float32),
           pltpu.VMEM((N_NODES, 4), jnp.float32)],
    )(pos, emb, wemb, bemb, kvec, *layer_params)
    return out[:, :3]


def kernel(positions, node_features, global_features, params):
    node_attrs = jax.nn.one_hot(node_features - 1, NUM_SPECIES)
    t = jnp.tile(global_features[None, :], (N_NODES, 1))
    emb = jnp.concatenate(
        [node_attrs, t,
         jnp.zeros((N_NODES, 3), jnp.float32)], axis=-1)
    wemb = jnp.concatenate(
        [params['W_emb'], jnp.zeros((3, F), jnp.float32)], axis=0)
    bemb = params['b_emb'][None, :]
    layer_params = []
    for p in params['layers']:
        layer_params += [
            p['Wr1'], p['br1'][None, :], p['Wr2'], p['Wsh'], p['Ws'],
            p['Wv'], p['P1'], p['P2'], p['P3'], p['Pv'], p['Wg'],
            p['wvec'][None, :],
        ]
    kvec = jnp.asarray(
        (np.arange(1, N_RADIAL + 1) * np.pi / R_MAX)[None, :],
        jnp.float32)
    return _run(positions.astype(jnp.float32), emb, wemb, bemb, kvec,
                *layer_params)
